# baseline (speedup 1.0000x reference)
import functools

import jax
import jax.numpy as jnp
from jax import lax
from jax.experimental import pallas as pl
from jax.experimental.pallas import tpu as pltpu
from jax.experimental.pallas import tpu_sc as plsc

EMBED = 64
SCALE = 8.0

_info = plsc.get_sparse_core_info()
_NC, _NS, _L = _info.num_cores, _info.num_subcores, _info.num_lanes
_NW = _NC * _NS

_VOCAB_BLK = 512


def _worker_id():
    return lax.axis_index("s") * _NC + lax.axis_index("c")


def _mesh():
    return plsc.VectorSubcoreMesh(core_axis_name="c", subcore_axis_name="s")


def _rowmajor_table(table_t, tail128):
    emb, vocab = table_t.shape
    n_full = vocab // _VOCAB_BLK
    rem = vocab - n_full * _VOCAB_BLK
    n_iter = (n_full + _NW - 1) // _NW

    @functools.partial(
        pl.kernel,
        out_type=jax.ShapeDtypeStruct((vocab * emb,), jnp.float32),
        mesh=_mesh(),
        scratch_types=[
            pltpu.VMEM((emb, _VOCAB_BLK), jnp.float32),
            pltpu.VMEM((_VOCAB_BLK * emb,), jnp.float32),
            pltpu.SemaphoreType.DMA,
        ],
        compiler_params=pltpu.CompilerParams(
            use_tc_tiling_on_sc=True, needs_layout_passes=False
        ),
    )
    def k1(tt_hbm, tail_hbm, out_hbm, in_v, t_v, sem):
        wid = _worker_id()
        lane = lax.iota(jnp.int32, _L)

        def blk_body(g, carry):
            b = wid + g * _NW

            @pl.when(b < n_full)
            def _full():
                c0 = b * _VOCAB_BLK
                pltpu.async_copy(
                    tt_hbm.at[:, pl.ds(c0, _VOCAB_BLK)], in_v, sem
                ).wait()

                def col_body(v, carry2):
                    for kk in range(emb // _L):
                        vals = plsc.load_gather(
                            in_v, [lane + kk * _L, lane * 0 + v]
                        )
                        t_v[pl.ds(v * emb + kk * _L, _L)] = vals
                    return carry2

                lax.fori_loop(0, _VOCAB_BLK, col_body, 0)
                pltpu.sync_copy(
                    t_v, out_hbm.at[pl.ds(c0 * emb, _VOCAB_BLK * emb)]
                )

            return carry

        lax.fori_loop(0, n_iter, blk_body, 0)

        if rem:
            @pl.when(wid == _NW - 1)
            def _partial():
                pltpu.async_copy(
                    tail_hbm, in_v.at[:, pl.ds(0, 2 * emb)], sem
                ).wait()

                def row_body(r, carry2):
                    for kk in range(emb // _L):
                        t_v[pl.ds(r * emb + kk * _L, _L)] = in_v[
                            r, pl.ds(kk * _L, _L)
                        ]
                    return carry2

                lax.fori_loop(0, rem, row_body, 0)
                pltpu.sync_copy(
                    t_v.at[pl.ds(0, rem * emb)],
                    out_hbm.at[pl.ds(n_full * _VOCAB_BLK * emb, rem * emb)],
                )

    return k1(table_t, tail128)


def _gather_scaled(xt_flat, table_rm, n_rows, row_len):
    blk = 128
    n_it = n_rows // blk

    @functools.partial(
        pl.kernel,
        out_type=jax.ShapeDtypeStruct(
            (row_len, EMBED // 8, n_it, 8, blk), jnp.float32
        ),
        mesh=_mesh(),
        scratch_types=[
            pltpu.VMEM((blk,), jnp.int32),
            pltpu.VMEM((blk, EMBED), jnp.float32),
            pltpu.VMEM((EMBED // 8, 8, blk), jnp.float32),
            pltpu.SemaphoreType.DMA,
        ],
        compiler_params=pltpu.CompilerParams(
            use_tc_tiling_on_sc=False, needs_layout_passes=False
        ),
    )
    def k2(xt_hbm, tbl_hbm, z_hbm, idx_v, rows_v, t_v, sem):
        wid = _worker_id()
        lane = lax.iota(jnp.int32, _L)

        def j_body(j, carry):
            pltpu.sync_copy(
                xt_hbm.at[pl.ds(j * n_rows + wid * blk, blk)], idx_v
            )
            pltpu.async_copy(tbl_hbm.at[idx_v], rows_v, sem).wait()

            def d_body(d, carry2):
                for kk in range(blk // _L):
                    vals = plsc.load_gather(
                        rows_v, [lane + kk * _L, lane * 0 + d]
                    )
                    t_v[d // 8, d % 8, pl.ds(kk * _L, _L)] = vals * SCALE
                return carry2

            lax.fori_loop(0, EMBED, d_body, 0)
            pltpu.sync_copy(t_v, z_hbm.at[j, :, wid])
            return carry

        lax.fori_loop(0, row_len, j_body, 0)

    return k2(xt_flat, table_rm)


@functools.partial(jax.jit, static_argnums=(3, 4))
def _lookup(xt_flat, table_t, tail128, n_rows, row_len):
    flat_rm = _rowmajor_table(table_t, tail128)
    table_rm = flat_rm.reshape(table_t.shape[1], table_t.shape[0])
    z = _gather_scaled(xt_flat, table_rm, n_rows, row_len)
    out = z.transpose(2, 4, 0, 1, 3).reshape(n_rows, row_len, EMBED)
    return out


def kernel(x, embedding_table):
    n_rows, row_len = x.shape
    vocab = embedding_table.shape[0]
    rem = vocab % _VOCAB_BLK
    xt_flat = x.T.reshape(-1).astype(jnp.int32)
    tail = embedding_table[vocab - rem:, :]
    tail128 = jnp.pad(tail, ((0, 0), (0, EMBED)))
    return _lookup(xt_flat, embedding_table.T, tail128, n_rows, row_len)
